# Initial kernel scaffold; baseline (speedup 1.0000x reference)
import jax
import jax.numpy as jnp
from jax.experimental import pallas as pl


def kernel(x, octave_w, pitch_w, short_dur_w, medium_dur_w, long_dur_w, velocity_w, short_shift_w, long_shift_w):
    raise NotImplementedError("write your pallas kernel here")


# speedup vs baseline: 13.1763x; 13.1763x over previous
import functools

import jax
import jax.numpy as jnp
from jax import lax
from jax.experimental import pallas as pl
from jax.experimental.pallas import tpu as pltpu
from jax.experimental.pallas import tpu_sc as plsc

L = 16
NW = 32
M = 1024 * 64 * 16
PW = M // NW
C = 512
NCH = PW // C
XC = C * 8
OC = C * 64

_PAD_VREGS = (8, 12, 16, 20, 24, 28)


def _vgather(x, idx):
    dnums = lax.GatherDimensionNumbers(
        offset_dims=(), collapsed_slice_dims=(0,), start_index_map=(0,))
    return lax.gather(x, idx[:, None], dnums, slice_sizes=(1,),
                      mode=lax.GatherScatterMode.PROMISE_IN_BOUNDS)


def _rsqrt_nr(s):
    y = lax.bitcast_convert_type(
        jnp.int32(0x5F3759DF) - lax.shift_right_logical(
            lax.bitcast_convert_type(s, jnp.int32), 1),
        jnp.float32,
    )
    for _ in range(4):
        y = y * (1.5 - 0.5 * s * y * y)
    return y


_mesh = plsc.VectorSubcoreMesh(core_axis_name="c", subcore_axis_name="s")


@functools.partial(
    pl.kernel,
    mesh=_mesh,
    compiler_params=pltpu.CompilerParams(needs_layout_passes=False),
    out_type=jax.ShapeDtypeStruct((M * 64,), jnp.float32),
    scratch_types=[
        pltpu.VMEM((512,), jnp.float32),
        pltpu.VMEM((XC,), jnp.int32),
        pltpu.VMEM((OC,), jnp.float32),
    ],
)
def _embed_sc(x_hbm, t0, t1, t2, t3, t4, t5, t6, t7, out_hbm, lut_v, x_v, o_v):
    wid = lax.axis_index("s") * 2 + lax.axis_index("c")
    iota = lax.iota(jnp.int32, L)

    for k, t in enumerate((t0, t1, t2, t3, t4, t5, t6, t7)):
        pltpu.sync_copy(t, lut_v.at[pl.ds(k * 64, 64)])

    padmask = jnp.where(iota < 8, 0.0, 1.0)
    for r in range(32):
        v = lut_v[pl.ds(r * L, L)]
        if r in _PAD_VREGS:
            v = v * padmask
        s = v * v
        for sh in (1, 2, 4):
            s = s + _vgather(s, iota ^ sh)
        s = jnp.maximum(s, 1e-24)
        scale = jnp.minimum(_rsqrt_nr(s), 1.0)
        lut_v[pl.ds(r * L, L)] = v * scale

    xbase = wid * (PW * 8)
    obase = wid * (PW * 64)
    ioffs = (iota & 7) * 64
    opos0 = iota * 8

    def chunk_body(ci, _):
        pltpu.sync_copy(x_hbm.at[pl.ds(xbase + ci * XC, XC)], x_v)

        def t_body(t, _):
            xv = x_v[pl.ds(t * L, L)]
            base = ioffs + xv * 8
            op = opos0 + t * 128
            for d in range(8):
                vals = plsc.load_gather(lut_v, [base + d])
                plsc.store_scatter(o_v, [op + d], vals)
            return 0

        lax.fori_loop(0, XC // L, t_body, 0)
        pltpu.sync_copy(o_v, out_hbm.at[pl.ds(obase + ci * OC, OC)])
        return 0

    lax.fori_loop(0, NCH, chunk_body, 0)


def kernel(x, octave_w, pitch_w, short_dur_w, medium_dur_w, long_dur_w,
           velocity_w, short_shift_w, long_shift_w):
    xf = x.reshape(-1).astype(jnp.int32)
    tabs = [w[:8].reshape(-1) for w in (octave_w, pitch_w, short_dur_w,
                                        medium_dur_w, long_dur_w, velocity_w,
                                        short_shift_w, long_shift_w)]
    out = _embed_sc(xf, *tabs)
    return out.reshape(1024, 64, 16, 64)


# speedup vs baseline: 18.4601x; 1.4010x over previous
import functools

import jax
import jax.numpy as jnp
from jax import lax
from jax.experimental import pallas as pl
from jax.experimental.pallas import tpu as pltpu
from jax.experimental.pallas import tpu_sc as plsc

L = 16
NW = 32
M = 1024 * 64 * 16
PW = M // NW
C = 512
NCH = PW // C
XC = C * 8
OC = C * 64

_PAD_VREGS = (8, 12, 16, 20, 24, 28)


def _vgather(x, idx):
    dnums = lax.GatherDimensionNumbers(
        offset_dims=(), collapsed_slice_dims=(0,), start_index_map=(0,))
    return lax.gather(x, idx[:, None], dnums, slice_sizes=(1,),
                      mode=lax.GatherScatterMode.PROMISE_IN_BOUNDS)


def _rsqrt_nr(s):
    y = lax.bitcast_convert_type(
        jnp.int32(0x5F3759DF) - lax.shift_right_logical(
            lax.bitcast_convert_type(s, jnp.int32), 1),
        jnp.float32,
    )
    for _ in range(4):
        y = y * (1.5 - 0.5 * s * y * y)
    return y


_mesh = plsc.VectorSubcoreMesh(core_axis_name="c", subcore_axis_name="s")


@functools.partial(
    pl.kernel,
    mesh=_mesh,
    compiler_params=pltpu.CompilerParams(needs_layout_passes=False),
    out_type=jax.ShapeDtypeStruct((M * 64,), jnp.float32),
    scratch_types=[
        pltpu.VMEM((512,), jnp.float32),
        pltpu.VMEM((XC,), jnp.int32),
        pltpu.VMEM((OC,), jnp.float32),
    ],
)
def _embed_sc(x_hbm, t0, t1, t2, t3, t4, t5, t6, t7, out_hbm, lut_v, x_v, o_v):
    wid = lax.axis_index("s") * 2 + lax.axis_index("c")
    iota = lax.iota(jnp.int32, L)

    for k, t in enumerate((t0, t1, t2, t3, t4, t5, t6, t7)):
        pltpu.sync_copy(t, lut_v.at[pl.ds(k * 64, 64)])

    padmask = jnp.where(iota < 8, 0.0, 1.0)
    for r in range(32):
        v = lut_v[pl.ds(r * L, L)]
        if r in _PAD_VREGS:
            v = v * padmask
        s = v * v
        for sh in (1, 2, 4):
            s = s + _vgather(s, iota ^ sh)
        s = jnp.maximum(s, 1e-24)
        scale = jnp.minimum(_rsqrt_nr(s), 1.0)
        lut_v[pl.ds(r * L, L)] = v * scale

    xbase = wid * (PW * 8)
    obase = wid * (PW * 64)
    ioffs = (iota & 7) * 64
    opos0 = iota * 8

    def chunk_body(ci, _):
        pltpu.sync_copy(x_hbm.at[pl.ds(xbase + ci * XC, XC)], x_v)

        @plsc.parallel_loop(0, XC // L, unroll=8)
        def t_body(t):
            xv = x_v[pl.ds(t * L, L)]
            base = ioffs + xv * 8
            op = opos0 + t * 128
            for d in range(8):
                vals = plsc.load_gather(lut_v, [base + d])
                plsc.store_scatter(o_v, [op + d], vals)

        pltpu.sync_copy(o_v, out_hbm.at[pl.ds(obase + ci * OC, OC)])
        return 0

    lax.fori_loop(0, NCH, chunk_body, 0)


def kernel(x, octave_w, pitch_w, short_dur_w, medium_dur_w, long_dur_w,
           velocity_w, short_shift_w, long_shift_w):
    xf = x.reshape(-1).astype(jnp.int32)
    tabs = [w[:8].reshape(-1) for w in (octave_w, pitch_w, short_dur_w,
                                        medium_dur_w, long_dur_w, velocity_w,
                                        short_shift_w, long_shift_w)]
    out = _embed_sc(xf, *tabs)
    return out.reshape(1024, 64, 16, 64)


# speedup vs baseline: 22.1742x; 1.2012x over previous
import functools

import jax
import jax.numpy as jnp
from jax import lax
from jax.experimental import pallas as pl
from jax.experimental.pallas import tpu as pltpu
from jax.experimental.pallas import tpu_sc as plsc

L = 16
NW = 32
M = 1024 * 64 * 16
PW = M // NW
C = 512
NCH = PW // C
XC = C * 8
OC = C * 64

_PAD_VREGS = (8, 12, 16, 20, 24, 28)


def _vgather(x, idx):
    dnums = lax.GatherDimensionNumbers(
        offset_dims=(), collapsed_slice_dims=(0,), start_index_map=(0,))
    return lax.gather(x, idx[:, None], dnums, slice_sizes=(1,),
                      mode=lax.GatherScatterMode.PROMISE_IN_BOUNDS)


def _rsqrt_nr(s):
    y = lax.bitcast_convert_type(
        jnp.int32(0x5F3759DF) - lax.shift_right_logical(
            lax.bitcast_convert_type(s, jnp.int32), 1),
        jnp.float32,
    )
    for _ in range(4):
        y = y * (1.5 - 0.5 * s * y * y)
    return y


_mesh = plsc.VectorSubcoreMesh(core_axis_name="c", subcore_axis_name="s")


@functools.partial(
    pl.kernel,
    mesh=_mesh,
    compiler_params=pltpu.CompilerParams(needs_layout_passes=False),
    out_type=jax.ShapeDtypeStruct((M * 64,), jnp.float32),
    scratch_types=[
        pltpu.VMEM((512,), jnp.float32),
        pltpu.VMEM((1024,), jnp.float32),
        pltpu.VMEM((XC,), jnp.int32),
        pltpu.VMEM((OC,), jnp.float32),
    ],
)
def _embed_sc(x_hbm, t0, t1, t2, t3, t4, t5, t6, t7, out_hbm, raw_v, lut_v,
              x_v, o_v):
    wid = lax.axis_index("s") * 2 + lax.axis_index("c")
    iota = lax.iota(jnp.int32, L)

    for k, t in enumerate((t0, t1, t2, t3, t4, t5, t6, t7)):
        pltpu.sync_copy(t, raw_v.at[pl.ds(k * 64, 64)])

    padmask = jnp.where(iota < 8, 0.0, 1.0)
    lane8 = iota & 7
    for r in range(32):
        v = raw_v[pl.ds(r * L, L)]
        if r in _PAD_VREGS:
            v = v * padmask
        s = v * v
        for sh in (1, 2, 4):
            s = s + _vgather(s, iota ^ sh)
        s = jnp.maximum(s, 1e-24)
        scale = jnp.minimum(_rsqrt_nr(s), 1.0)
        sv = v * scale
        lut_v[pl.ds((2 * r) * L, L)] = _vgather(sv, lane8)
        lut_v[pl.ds((2 * r + 1) * L, L)] = _vgather(sv, lane8 + 8)

    xbase = wid * (PW * 8)
    obase = wid * (PW * 64)
    ioffs = lane8 * 128
    pats = [jnp.where(iota < 8, 2 * r, 2 * r + 1) for r in range(8)]

    def chunk_body(ci, _):
        pltpu.sync_copy(x_hbm.at[pl.ds(xbase + ci * XC, XC)], x_v)

        @plsc.parallel_loop(0, XC // L, unroll=8)
        def t_body(t):
            xv = x_v[pl.ds(t * L, L)]
            base = ioffs + xv * 16
            for r in range(8):
                idx = _vgather(base, pats[r]) + iota
                o_v[pl.ds(t * 128 + r * L, L)] = plsc.load_gather(lut_v, [idx])

        pltpu.sync_copy(o_v, out_hbm.at[pl.ds(obase + ci * OC, OC)])
        return 0

    lax.fori_loop(0, NCH, chunk_body, 0)


def kernel(x, octave_w, pitch_w, short_dur_w, medium_dur_w, long_dur_w,
           velocity_w, short_shift_w, long_shift_w):
    xf = x.reshape(-1).astype(jnp.int32)
    tabs = [w[:8].reshape(-1) for w in (octave_w, pitch_w, short_dur_w,
                                        medium_dur_w, long_dur_w, velocity_w,
                                        short_shift_w, long_shift_w)]
    out = _embed_sc(xf, *tabs)
    return out.reshape(1024, 64, 16, 64)


# speedup vs baseline: 24.4181x; 1.1012x over previous
import functools

import jax
import jax.numpy as jnp
from jax import lax
from jax.experimental import pallas as pl
from jax.experimental.pallas import tpu as pltpu
from jax.experimental.pallas import tpu_sc as plsc

L = 16
NW = 32
M = 1024 * 64 * 16
PW = M // NW
C = 512
NCH = PW // C
XC = C * 8
OC = C * 64

_PAD_VREGS = (8, 12, 16, 20, 24, 28)


def _vgather(x, idx):
    dnums = lax.GatherDimensionNumbers(
        offset_dims=(), collapsed_slice_dims=(0,), start_index_map=(0,))
    return lax.gather(x, idx[:, None], dnums, slice_sizes=(1,),
                      mode=lax.GatherScatterMode.PROMISE_IN_BOUNDS)


def _rsqrt_nr(s):
    y = lax.bitcast_convert_type(
        jnp.int32(0x5F3759DF) - lax.shift_right_logical(
            lax.bitcast_convert_type(s, jnp.int32), 1),
        jnp.float32,
    )
    for _ in range(4):
        y = y * (1.5 - 0.5 * s * y * y)
    return y


_mesh = plsc.VectorSubcoreMesh(core_axis_name="c", subcore_axis_name="s")


@functools.partial(
    pl.kernel,
    mesh=_mesh,
    compiler_params=pltpu.CompilerParams(needs_layout_passes=False),
    out_type=jax.ShapeDtypeStruct((M * 64,), jnp.float32),
    scratch_types=[
        pltpu.VMEM((512,), jnp.float32),
        pltpu.VMEM((1024,), jnp.float32),
        pltpu.VMEM((XC,), jnp.int32),
        pltpu.VMEM((XC,), jnp.int32),
        pltpu.VMEM((OC,), jnp.float32),
        pltpu.VMEM((OC,), jnp.float32),
        pltpu.SemaphoreType.DMA,
        pltpu.SemaphoreType.DMA,
        pltpu.SemaphoreType.DMA,
        pltpu.SemaphoreType.DMA,
    ],
)
def _embed_sc(x_hbm, t0, t1, t2, t3, t4, t5, t6, t7, out_hbm, raw_v, lut_v,
              x_v0, x_v1, o_v0, o_v1, in_s0, in_s1, out_s0, out_s1):
    wid = lax.axis_index("s") * 2 + lax.axis_index("c")
    iota = lax.iota(jnp.int32, L)

    for k, t in enumerate((t0, t1, t2, t3, t4, t5, t6, t7)):
        pltpu.sync_copy(t, raw_v.at[pl.ds(k * 64, 64)])

    padmask = jnp.where(iota < 8, 0.0, 1.0)
    lane8 = iota & 7
    for r in range(32):
        v = raw_v[pl.ds(r * L, L)]
        if r in _PAD_VREGS:
            v = v * padmask
        s = v * v
        for sh in (1, 2, 4):
            s = s + _vgather(s, iota ^ sh)
        s = jnp.maximum(s, 1e-24)
        scale = jnp.minimum(_rsqrt_nr(s), 1.0)
        sv = v * scale
        lut_v[pl.ds((2 * r) * L, L)] = _vgather(sv, lane8)
        lut_v[pl.ds((2 * r + 1) * L, L)] = _vgather(sv, lane8 + 8)

    xbase = wid * (PW * 8)
    obase = wid * (PW * 64)
    ioffs = lane8 * 128
    pats = [jnp.where(iota < 8, 2 * r, 2 * r + 1) for r in range(8)]

    in_sems = (in_s0, in_s1)
    out_sems = (out_s0, out_s1)
    x_bufs = (x_v0, x_v1)
    o_bufs = (o_v0, o_v1)

    def x_copy(ci, b):
        return pltpu.make_async_copy(
            x_hbm.at[pl.ds(xbase + ci * XC, XC)], x_bufs[b], in_sems[b])

    def o_copy(ci, b):
        return pltpu.make_async_copy(
            o_bufs[b], out_hbm.at[pl.ds(obase + ci * OC, OC)], out_sems[b])

    x_copy(0, 0).start()

    def chunk_group(cg, _):
        for b in range(2):
            ci = 2 * cg + b
            x_copy(ci, b).wait()

            @pl.when(ci + 1 < NCH)
            def _():
                x_copy(ci + 1, 1 - b).start()

            @pl.when(ci >= 2)
            def _():
                o_copy(ci - 2, b).wait()

            ob = o_bufs[b]
            xb = x_bufs[b]

            @plsc.parallel_loop(0, XC // L, unroll=8)
            def t_body(t):
                xv = xb[pl.ds(t * L, L)]
                base = ioffs + xv * 16
                for r in range(8):
                    idx = _vgather(base, pats[r]) + iota
                    ob[pl.ds(t * 128 + r * L, L)] = plsc.load_gather(
                        lut_v, [idx])

            o_copy(ci, b).start()
        return 0

    lax.fori_loop(0, NCH // 2, chunk_group, 0)
    o_copy(NCH - 2, 0).wait()
    o_copy(NCH - 1, 1).wait()


def kernel(x, octave_w, pitch_w, short_dur_w, medium_dur_w, long_dur_w,
           velocity_w, short_shift_w, long_shift_w):
    xf = x.reshape(-1).astype(jnp.int32)
    tabs = [w[:8].reshape(-1) for w in (octave_w, pitch_w, short_dur_w,
                                        medium_dur_w, long_dur_w, velocity_w,
                                        short_shift_w, long_shift_w)]
    out = _embed_sc(xf, *tabs)
    return out.reshape(1024, 64, 16, 64)
